# Initial kernel scaffold; baseline (speedup 1.0000x reference)
import jax
import jax.numpy as jnp
from jax.experimental import pallas as pl


def kernel(user_emb, item_emb, a_vals, s_vals, fc1_w, fc1_b, fc2_w, fc2_b, fc3_w, fc3_b, edge_index_a, edge_index_s):
    raise NotImplementedError("write your pallas kernel here")


# speedup vs baseline: 4.1955x; 4.1955x over previous
import functools

import jax
import jax.numpy as jnp
from jax import lax
from jax.experimental import pallas as pl
from jax.experimental.pallas import tpu as pltpu
from jax.experimental.pallas import tpu_sc as plsc

N_U = 2500
N_I = 7500
N = N_U + N_I
D = 128
E_A = 320000
E_S = 80000
K = 128
NC = 2
NS = 16
NW = NC * NS
CA = E_A // K
CS = E_S // K
RA0, RA_LAST = 624, N - 15 * 624
RS0, RS_LAST = 152, N_U - 15 * 152

_mesh = plsc.VectorSubcoreMesh(core_axis_name="c", subcore_axis_name="s")


@functools.partial(
    pl.kernel,
    out_type=(
        jax.ShapeDtypeStruct((NC, N, D), jnp.float32),
        jax.ShapeDtypeStruct((NC, N_U, D), jnp.float32),
    ),
    mesh=_mesh,
    scratch_types=[
        pltpu.VMEM_SHARED((N, D), jnp.float32),
        pltpu.VMEM_SHARED((N_U, D), jnp.float32),
        pltpu.VMEM((K,), jnp.int32),
        pltpu.VMEM((K,), jnp.int32),
        pltpu.VMEM((K,), jnp.float32),
        pltpu.VMEM((K, D), jnp.float32),
        pltpu.SemaphoreType.DMA,
    ],
)
def _sc_spmm(x_hbm, sx_hbm, srca_hbm, dsta_hbm, va_hbm, srcs_hbm, dsts_hbm,
             vs_hbm, pa_hbm, ps_hbm, acc_a, acc_s, srcv, dstv, valv, rows,
             sem):
    cid = lax.axis_index("c")
    sid = lax.axis_index("s")
    wid = sid * NC + cid

    zero = jnp.zeros((16,), jnp.float32)

    def _zrow(k, _):
        for j in range(D // 16):
            rows[k, pl.ds(j * 16, 16)] = zero
        return 0

    lax.fori_loop(0, K, _zrow, 0)

    def _fill(dst, base, n):
        full, rem = n // K, n % K
        for r in range(full):
            pltpu.sync_copy(rows.at[pl.ds(0, K)],
                            dst.at[pl.ds(base + r * K, K)])
        if rem:
            pltpu.sync_copy(rows.at[pl.ds(0, rem)],
                            dst.at[pl.ds(base + full * K, rem)])

    @pl.when(sid < 15)
    def _():
        _fill(acc_a, sid * RA0, RA0)
        _fill(acc_s, sid * RS0, RS0)

    @pl.when(sid == 15)
    def _():
        _fill(acc_a, 15 * RA0, RA_LAST)
        _fill(acc_s, 15 * RS0, RS_LAST)

    plsc.subcore_barrier()

    def _edge_chunk(base, src_hbm, dst_hbm, v_hbm, x_ref, acc):
        pltpu.sync_copy(src_hbm.at[pl.ds(base, K)], srcv)
        pltpu.sync_copy(dst_hbm.at[pl.ds(base, K)], dstv)
        pltpu.sync_copy(v_hbm.at[pl.ds(base, K)], valv)
        pltpu.async_copy(x_ref.at[srcv], rows, sem).wait()

        def _scale(g, _):
            vv = valv[pl.ds(g * 16, 16)]
            for e in range(16):
                v = vv[e]
                r = g * 16 + e
                for j in range(D // 16):
                    rows[r, pl.ds(j * 16, 16)] = rows[r, pl.ds(j * 16, 16)] * v
            return 0

        lax.fori_loop(0, K // 16, _scale, 0)
        pltpu.sync_copy(rows, acc.at[dstv], add=True)

    na = (CA - 1 - wid) // NW + 1

    def _a_body(i, _):
        _edge_chunk((wid + i * NW) * K, srca_hbm, dsta_hbm, va_hbm,
                    x_hbm, acc_a)
        return 0

    lax.fori_loop(0, na, _a_body, 0)

    ns = (CS - 1 - wid) // NW + 1

    def _s_body(i, _):
        _edge_chunk((wid + i * NW) * K, srcs_hbm, dsts_hbm, vs_hbm,
                    sx_hbm, acc_s)
        return 0

    lax.fori_loop(0, ns, _s_body, 0)

    plsc.subcore_barrier()

    @pl.when(sid < 15)
    def _():
        pltpu.sync_copy(acc_a.at[pl.ds(sid * RA0, RA0)],
                        pa_hbm.at[cid, pl.ds(sid * RA0, RA0)])
        pltpu.sync_copy(acc_s.at[pl.ds(sid * RS0, RS0)],
                        ps_hbm.at[cid, pl.ds(sid * RS0, RS0)])

    @pl.when(sid == 15)
    def _():
        pltpu.sync_copy(acc_a.at[pl.ds(15 * RA0, RA_LAST)],
                        pa_hbm.at[cid, pl.ds(15 * RA0, RA_LAST)])
        pltpu.sync_copy(acc_s.at[pl.ds(15 * RS0, RS_LAST)],
                        ps_hbm.at[cid, pl.ds(15 * RS0, RS_LAST)])


def _mish(x):
    sp = jnp.maximum(x, 0.0) + jnp.log(1.0 + jnp.exp(-jnp.abs(x)))
    return x * jnp.tanh(sp)


def _fusion(u, s, f1w, f1b, f2w, f2b, f3w, f3b):
    c = jnp.concatenate([u, s, u * s], axis=1)
    t1 = _mish(jnp.dot(c, f1w, preferred_element_type=jnp.float32) + f1b)
    t2 = _mish(jnp.dot(t1, f2w, preferred_element_type=jnp.float32) + f2b)
    t3 = jnp.dot(t2, f3w, preferred_element_type=jnp.float32) + f3b
    return t3 / jnp.sqrt(jnp.sum(t3 * t3))


def _tc1_body(pa, ps, f1w, f1b, f2w, f2b, f3w, f3b, ego_out):
    a = pa[0] + pa[1]
    s = ps[0] + ps[1]
    u = a[:N_U]
    ego_out[pl.ds(0, N_U), :] = _fusion(u, s, f1w[...], f1b[...], f2w[...],
                                        f2b[...], f3w[...], f3b[...])
    ego_out[pl.ds(N_U, N_I), :] = a[N_U:]


def _tc2_body(qa, qs, ue, ie, ego1, f1w, f1b, f2w, f2b, f3w, f3b,
              user_out, item_out):
    a = qa[0] + qa[1]
    s = qs[0] + qs[1]
    t3n = _fusion(a[:N_U], s, f1w[...], f1b[...], f2w[...], f2b[...],
                  f3w[...], f3b[...])
    user_out[...] = (ue[...] + ego1[pl.ds(0, N_U), :] + t3n) * (1.0 / 3.0)
    item_out[...] = (ie[...] + ego1[pl.ds(N_U, N_I), :] + a[N_U:]) * (1.0 / 3.0)


_tc1 = pl.pallas_call(
    _tc1_body,
    out_shape=jax.ShapeDtypeStruct((N, D), jnp.float32),
)

_tc2 = pl.pallas_call(
    _tc2_body,
    out_shape=(
        jax.ShapeDtypeStruct((N_U, D), jnp.float32),
        jax.ShapeDtypeStruct((N_I, D), jnp.float32),
    ),
)


def kernel(user_emb, item_emb, a_vals, s_vals, fc1_w, fc1_b, fc2_w, fc2_b,
           fc3_w, fc3_b, edge_index_a, edge_index_s):
    x0 = jnp.concatenate([user_emb, item_emb], axis=0)
    src_a, dst_a = edge_index_a[0], edge_index_a[1]
    src_s, dst_s = edge_index_s[0], edge_index_s[1]

    pa, ps = _sc_spmm(x0, user_emb, src_a, dst_a, a_vals, src_s, dst_s,
                      s_vals)
    ego1 = _tc1(pa, ps, fc1_w, fc1_b, fc2_w, fc2_b, fc3_w, fc3_b)
    qa, qs = _sc_spmm(ego1, ego1[:N_U], src_a, dst_a, a_vals, src_s, dst_s,
                      s_vals)
    user_out, item_out = _tc2(qa, qs, user_emb, item_emb, ego1, fc1_w, fc1_b,
                              fc2_w, fc2_b, fc3_w, fc3_b)
    return user_out, item_out
